# baseline (speedup 1.0000x reference)
import dataclasses
import functools

import jax
import jax.numpy as jnp
from jax.experimental import pallas as pl
from jax.experimental.pallas import tpu as pltpu
from jax.experimental.pallas import tpu_sc as plsc

N = 4096
D = 32
K = 16
KP1 = K + 1
NH = N // 2
BR = 256
CH = 256
GROUPS = CH // K


def _knn_body(emb_r_ref, embt_ref, knn_d_ref, knn_i_ref, *, row_base):
    emb_r = emb_r_ref[...]
    embt = embt_ref[...]
    sq_col = jnp.sum(emb_r * emb_r, axis=1, keepdims=True)
    sq_row = jnp.sum(embt * embt, axis=0, keepdims=True)
    dots = jax.lax.dot_general(
        emb_r, embt, (((1,), (0,)), ((), ())),
        preferred_element_type=jnp.float32)
    d2 = jnp.maximum(sq_col + sq_row - 2.0 * dots, 0.0)

    iota = jax.lax.broadcasted_iota(jnp.int32, (BR, N), 1)
    vmask = jnp.int32(-4096)
    maxi = jnp.int32(0x7FFFFFFF)
    row_id = (row_base + pl.program_id(0) * BR
              + jax.lax.broadcasted_iota(jnp.int32, (BR, N), 0))
    key = (jax.lax.bitcast_convert_type(d2, jnp.int32) & vmask) | iota
    key = jnp.where(iota == row_id, maxi, key)
    q = N // 4
    qa, qb = key[:, :q], key[:, q:2 * q]
    qc, qd = key[:, 2 * q:3 * q], key[:, 3 * q:]
    m1, x1 = jnp.minimum(qa, qb), jnp.maximum(qa, qb)
    m2, x2 = jnp.minimum(qc, qd), jnp.maximum(qc, qd)
    r4 = jnp.minimum(m1, m2)
    s4 = jnp.where(m1 < m2, jnp.minimum(m2, x1), jnp.minimum(m1, x2))
    e = N // 8
    ra, rb = r4[:, :e], r4[:, e:]
    sa, sb = s4[:, :e], s4[:, e:]
    r = jnp.minimum(ra, rb)
    s = jnp.where(ra < rb, jnp.minimum(rb, sa), jnp.minimum(ra, sb))
    for t in range(K):
        m = jnp.min(r, axis=1, keepdims=True)
        knn_d_ref[:, t:t + 1] = jax.lax.bitcast_convert_type(
            m & vmask, jnp.float32)
        knn_i_ref[:, t:t + 1] = m & jnp.int32(0xFFF)
        if t < K - 1:
            r = jnp.where(r == m, jnp.where(s == m, maxi, s), r)


def _knn_call(emb_half, embt, row_base):
    return pl.pallas_call(
        functools.partial(_knn_body, row_base=row_base),
        grid=(NH // BR,),
        in_specs=[
            pl.BlockSpec((BR, D), lambda i: (i, 0)),
            pl.BlockSpec((D, N), lambda i: (0, 0)),
        ],
        out_specs=[
            pl.BlockSpec((BR, K), lambda i: (i, 0)),
            pl.BlockSpec((BR, K), lambda i: (i, 0)),
        ],
        out_shape=[
            jax.ShapeDtypeStruct((NH, K), jnp.float32),
            jax.ShapeDtypeStruct((NH, K), jnp.int32),
        ],
    )(emb_half, embt)


_GATHER_W = 256
DPAD = 128


def _sc_gather(emb_pad, idx_flat):
    mesh = plsc.VectorSubcoreMesh(core_axis_name="core",
                                  subcore_axis_name="subcore")
    cp = pltpu.CompilerParams()
    if "needs_layout_passes" in pltpu.CompilerParams.__dataclass_fields__:
        cp = dataclasses.replace(cp, needs_layout_passes=False)

    @functools.partial(
        pl.kernel,
        out_type=jax.ShapeDtypeStruct((NH * K, DPAD), jnp.float32),
        mesh=mesh,
        compiler_params=cp,
    )
    def gather_kernel(x_hbm, i_hbm, o_hbm):
        def body(i_vmem, o_vmem):
            pltpu.sync_copy(x_hbm.at[i_vmem.at[0]], o_vmem)

        pltpu.emit_pipeline(
            body,
            grid=(NH * K // _GATHER_W,),
            in_specs=[pl.BlockSpec((1, _GATHER_W), index_map=lambda i: (0, i))],
            out_specs=[pl.BlockSpec((_GATHER_W, DPAD), index_map=lambda i: (i, 0))],
            core_axis_name=("core", "subcore"),
            dimension_semantics=(pltpu.PARALLEL,),
        )(i_hbm, o_hbm)

    return gather_kernel(emb_pad, idx_flat)


def _inter_body(g_ref, knn_d_ref, refc_ref, refd_ref,
                curv_ref, knnsum_ref, refdsum_ref):
    step = pl.program_id(0)

    @pl.when(step == 0)
    def _init():
        curv_ref[...] = jnp.zeros_like(curv_ref)
        knnsum_ref[...] = jnp.zeros_like(knnsum_ref)
        refdsum_ref[...] = jnp.zeros_like(refdsum_ref)

    knn_d = jnp.sqrt(knn_d_ref[...] + 1e-12)
    a = jnp.mean(knn_d, axis=1, keepdims=True)
    refc = refc_ref[...]

    ri = jax.lax.broadcasted_iota(jnp.int32, (CH, CH), 0)
    ci = jax.lax.broadcasted_iota(jnp.int32, (CH, CH), 1)
    maskf = ((ri // K == ci // K) & (ci > ri)).astype(jnp.float32)
    gi = jax.lax.broadcasted_iota(jnp.int32, (GROUPS, CH), 0)
    gf = jax.lax.broadcasted_iota(jnp.int32, (GROUPS, CH), 1)
    pmat = (gf // K == gi).astype(jnp.float32)

    denom = K * (K - 1) / 2.0
    ones_row = jnp.ones((1, D), jnp.float32)
    ones_col = jnp.ones((CH, 1), jnp.float32)
    partial = jnp.zeros((1, 1), jnp.float32)
    for j in range(BR * K // CH):
        sub = g_ref[j * CH:(j + 1) * CH, :D]
        sub2 = sub * sub
        sq_col = jnp.sum(sub2, axis=1, keepdims=True)
        sq_row = jax.lax.dot_general(
            ones_row, sub2, (((1,), (1,)), ((), ())),
            preferred_element_type=jnp.float32)
        dots = jax.lax.dot_general(
            sub, sub, (((1,), (1,)), ((), ())),
            preferred_element_type=jnp.float32)
        d2 = sq_col + sq_row - 2.0 * dots
        dist = jnp.sqrt(jnp.maximum(d2, 0.0) + 1e-12) * maskf
        msum = jax.lax.dot_general(
            dist, ones_col, (((1,), (0,)), ((), ())),
            preferred_element_type=jnp.float32)
        gs = jax.lax.dot_general(
            pmat, msum, (((1,), (0,)), ((), ())),
            preferred_element_type=jnp.float32)
        b = gs / denom
        a_j = a[j * GROUPS:(j + 1) * GROUPS, :]
        refc_j = refc[j * GROUPS:(j + 1) * GROUPS, :]
        diff = b / (a_j + 1e-8) - refc_j
        partial = partial + jnp.sum(diff * diff, keepdims=True)

    curv_ref[...] += partial
    knnsum_ref[...] += jnp.sum(knn_d, keepdims=True)
    refdsum_ref[...] += jnp.sum(refd_ref[...], keepdims=True)


def _inter_call(gathered, knn_d, refc, refd):
    return pl.pallas_call(
        _inter_body,
        grid=(NH // BR,),
        in_specs=[
            pl.BlockSpec((BR * K, DPAD), lambda i: (i, 0)),
            pl.BlockSpec((BR, K), lambda i: (i, 0)),
            pl.BlockSpec((BR, 1), lambda i: (i, 0)),
            pl.BlockSpec((BR, K), lambda i: (i, 0)),
        ],
        out_specs=[
            pl.BlockSpec((1, 1), lambda i: (0, 0)),
            pl.BlockSpec((1, 1), lambda i: (0, 0)),
            pl.BlockSpec((1, 1), lambda i: (0, 0)),
        ],
        out_shape=[
            jax.ShapeDtypeStruct((1, 1), jnp.float32),
            jax.ShapeDtypeStruct((1, 1), jnp.float32),
            jax.ShapeDtypeStruct((1, 1), jnp.float32),
        ],
    )(gathered, knn_d, refc, refd)


def kernel(embeddings, ref_curvature, ref_distances):
    emb = embeddings.astype(jnp.float32)
    embt = jnp.transpose(emb)
    emb_pad = jnp.pad(emb, ((0, 0), (0, DPAD - D)))
    refc = ref_curvature.reshape(N, 1)

    knn = [_knn_call(emb[h * NH:(h + 1) * NH], embt, h * NH) for h in range(2)]
    gathered = [_sc_gather(emb_pad, knn[h][1].reshape(1, NH * K))
                for h in range(2)]
    curv_sq = jnp.float32(0.0)
    knn_sum = jnp.float32(0.0)
    refd_sum = jnp.float32(0.0)
    for h in range(2):
        c, ks, rs = _inter_call(
            gathered[h], knn[h][0], refc[h * NH:(h + 1) * NH],
            ref_distances[h * NH:(h + 1) * NH])
        curv_sq = curv_sq + c[0, 0]
        knn_sum = knn_sum + ks[0, 0]
        refd_sum = refd_sum + rs[0, 0]

    curvature_loss = curv_sq / N
    scale_loss = (knn_sum / (N * K) - refd_sum / (N * K)) ** 2
    return curvature_loss + 0.1 * scale_loss


# speedup vs baseline: 1.2610x; 1.2610x over previous
import dataclasses
import functools

import jax
import jax.numpy as jnp
from jax.experimental import pallas as pl
from jax.experimental.pallas import tpu as pltpu
from jax.experimental.pallas import tpu_sc as plsc

N = 4096
D = 32
K = 16
KP1 = K + 1
NH = N // 2
BR = 256
CH = 512
GROUPS = CH // K


def _knn_body(emb_r_ref, embt_ref, knn_d_ref, knn_i_ref, *, row_base):
    emb_r = emb_r_ref[...]
    embt = embt_ref[...]
    sq_col = jnp.sum(emb_r * emb_r, axis=1, keepdims=True)
    sq_row = jnp.sum(embt * embt, axis=0, keepdims=True)
    dots = jax.lax.dot_general(
        emb_r, embt, (((1,), (0,)), ((), ())),
        preferred_element_type=jnp.float32)
    d2 = jnp.maximum(sq_col + sq_row - 2.0 * dots, 0.0)

    iota = jax.lax.broadcasted_iota(jnp.int32, (BR, N), 1)
    vmask = jnp.int32(-4096)
    maxi = jnp.int32(0x7FFFFFFF)
    row_id = (row_base + pl.program_id(0) * BR
              + jax.lax.broadcasted_iota(jnp.int32, (BR, N), 0))
    key = (jax.lax.bitcast_convert_type(d2, jnp.int32) & vmask) | iota
    key = jnp.where(iota == row_id, maxi, key)
    q = N // 4
    qa, qb = key[:, :q], key[:, q:2 * q]
    qc, qd = key[:, 2 * q:3 * q], key[:, 3 * q:]
    m1, x1 = jnp.minimum(qa, qb), jnp.maximum(qa, qb)
    m2, x2 = jnp.minimum(qc, qd), jnp.maximum(qc, qd)
    r4 = jnp.minimum(m1, m2)
    s4 = jnp.where(m1 < m2, jnp.minimum(m2, x1), jnp.minimum(m1, x2))
    e = N // 8
    ra, rb = r4[:, :e], r4[:, e:]
    sa, sb = s4[:, :e], s4[:, e:]
    r8 = jnp.minimum(ra, rb)
    s8 = jnp.where(ra < rb, jnp.minimum(rb, sa), jnp.minimum(ra, sb))
    e2 = N // 16
    ra2, rb2 = r8[:, :e2], r8[:, e2:]
    sa2, sb2 = s8[:, :e2], s8[:, e2:]
    r = jnp.minimum(ra2, rb2)
    s = jnp.where(ra2 < rb2, jnp.minimum(rb2, sa2), jnp.minimum(ra2, sb2))
    for t in range(K):
        m = jnp.min(r, axis=1, keepdims=True)
        knn_d_ref[:, t:t + 1] = jax.lax.bitcast_convert_type(
            m & vmask, jnp.float32)
        knn_i_ref[:, t:t + 1] = m & jnp.int32(0xFFF)
        if t < K - 1:
            r = jnp.where(r == m, jnp.where(s == m, maxi, s), r)


def _knn_call(emb_half, embt, row_base):
    return pl.pallas_call(
        functools.partial(_knn_body, row_base=row_base),
        grid=(NH // BR,),
        in_specs=[
            pl.BlockSpec((BR, D), lambda i: (i, 0)),
            pl.BlockSpec((D, N), lambda i: (0, 0)),
        ],
        out_specs=[
            pl.BlockSpec((BR, K), lambda i: (i, 0)),
            pl.BlockSpec((BR, K), lambda i: (i, 0)),
        ],
        out_shape=[
            jax.ShapeDtypeStruct((NH, K), jnp.float32),
            jax.ShapeDtypeStruct((NH, K), jnp.int32),
        ],
    )(emb_half, embt)


_GATHER_W = 256
DPAD = 128


def _sc_gather(emb_pad, idx_flat):
    mesh = plsc.VectorSubcoreMesh(core_axis_name="core",
                                  subcore_axis_name="subcore")
    cp = pltpu.CompilerParams()
    if "needs_layout_passes" in pltpu.CompilerParams.__dataclass_fields__:
        cp = dataclasses.replace(cp, needs_layout_passes=False)

    @functools.partial(
        pl.kernel,
        out_type=jax.ShapeDtypeStruct((NH * K, DPAD), jnp.float32),
        mesh=mesh,
        compiler_params=cp,
    )
    def gather_kernel(x_hbm, i_hbm, o_hbm):
        def body(i_vmem, o_vmem):
            pltpu.sync_copy(x_hbm.at[i_vmem.at[0]], o_vmem)

        pltpu.emit_pipeline(
            body,
            grid=(NH * K // _GATHER_W,),
            in_specs=[pl.BlockSpec((1, _GATHER_W), index_map=lambda i: (0, i))],
            out_specs=[pl.BlockSpec((_GATHER_W, DPAD), index_map=lambda i: (i, 0))],
            core_axis_name=("core", "subcore"),
            dimension_semantics=(pltpu.PARALLEL,),
        )(i_hbm, o_hbm)

    return gather_kernel(emb_pad, idx_flat)


def _inter_body(g_ref, knn_d_ref, refc_ref, refd_ref,
                curv_ref, knnsum_ref, refdsum_ref):
    step = pl.program_id(0)

    @pl.when(step == 0)
    def _init():
        curv_ref[...] = jnp.zeros_like(curv_ref)
        knnsum_ref[...] = jnp.zeros_like(knnsum_ref)
        refdsum_ref[...] = jnp.zeros_like(refdsum_ref)

    knn_d = jnp.sqrt(knn_d_ref[...] + 1e-12)
    a = jnp.mean(knn_d, axis=1, keepdims=True)
    refc = refc_ref[...]

    ri = jax.lax.broadcasted_iota(jnp.int32, (CH, CH), 0)
    ci = jax.lax.broadcasted_iota(jnp.int32, (CH, CH), 1)
    mask = (ri // K == ci // K) & (ci > ri)
    gi = jax.lax.broadcasted_iota(jnp.int32, (GROUPS, CH), 0)
    gf = jax.lax.broadcasted_iota(jnp.int32, (GROUPS, CH), 1)
    pmat = (gf // K == gi).astype(jnp.float32)

    denom = K * (K - 1) / 2.0
    ones_row = jnp.ones((1, D), jnp.float32)
    partial = jnp.zeros((1, 1), jnp.float32)
    for j in range(BR * K // CH):
        sub = g_ref[j * CH:(j + 1) * CH, :D]
        sub2 = sub * sub
        sq_col = jnp.sum(sub2, axis=1, keepdims=True)
        sq_row = jax.lax.dot_general(
            ones_row, sub2, (((1,), (1,)), ((), ())),
            preferred_element_type=jnp.float32)
        dots = jax.lax.dot_general(
            sub, sub, (((1,), (1,)), ((), ())),
            preferred_element_type=jnp.float32)
        d2 = sq_col + sq_row - 2.0 * dots
        dist = jnp.sqrt(jnp.maximum(d2, 0.0) + 1e-12)
        msum = jnp.sum(jnp.where(mask, dist, 0.0), axis=1, keepdims=True)
        gs = jax.lax.dot_general(
            pmat, msum, (((1,), (0,)), ((), ())),
            preferred_element_type=jnp.float32)
        b = gs / denom
        a_j = a[j * GROUPS:(j + 1) * GROUPS, :]
        refc_j = refc[j * GROUPS:(j + 1) * GROUPS, :]
        diff = b / (a_j + 1e-8) - refc_j
        partial = partial + jnp.sum(diff * diff, keepdims=True)

    curv_ref[...] += partial
    knnsum_ref[...] += jnp.sum(knn_d, keepdims=True)
    refdsum_ref[...] += jnp.sum(refd_ref[...], keepdims=True)


def _inter_call(gathered, knn_d, refc, refd):
    return pl.pallas_call(
        _inter_body,
        grid=(NH // BR,),
        in_specs=[
            pl.BlockSpec((BR * K, DPAD), lambda i: (i, 0)),
            pl.BlockSpec((BR, K), lambda i: (i, 0)),
            pl.BlockSpec((BR, 1), lambda i: (i, 0)),
            pl.BlockSpec((BR, K), lambda i: (i, 0)),
        ],
        out_specs=[
            pl.BlockSpec((1, 1), lambda i: (0, 0)),
            pl.BlockSpec((1, 1), lambda i: (0, 0)),
            pl.BlockSpec((1, 1), lambda i: (0, 0)),
        ],
        out_shape=[
            jax.ShapeDtypeStruct((1, 1), jnp.float32),
            jax.ShapeDtypeStruct((1, 1), jnp.float32),
            jax.ShapeDtypeStruct((1, 1), jnp.float32),
        ],
    )(gathered, knn_d, refc, refd)


def kernel(embeddings, ref_curvature, ref_distances):
    emb = embeddings.astype(jnp.float32)
    embt = jnp.transpose(emb)
    emb_pad = jnp.pad(emb, ((0, 0), (0, DPAD - D)))
    refc = ref_curvature.reshape(N, 1)

    knn = [_knn_call(emb[h * NH:(h + 1) * NH], embt, h * NH) for h in range(2)]
    gathered = [_sc_gather(emb_pad, knn[h][1].reshape(1, NH * K))
                for h in range(2)]
    curv_sq = jnp.float32(0.0)
    knn_sum = jnp.float32(0.0)
    refd_sum = jnp.float32(0.0)
    for h in range(2):
        c, ks, rs = _inter_call(
            gathered[h], knn[h][0], refc[h * NH:(h + 1) * NH],
            ref_distances[h * NH:(h + 1) * NH])
        curv_sq = curv_sq + c[0, 0]
        knn_sum = knn_sum + ks[0, 0]
        refd_sum = refd_sum + rs[0, 0]

    curvature_loss = curv_sq / N
    scale_loss = (knn_sum / (N * K) - refd_sum / (N * K)) ** 2
    return curvature_loss + 0.1 * scale_loss


# speedup vs baseline: 1.4331x; 1.1365x over previous
import dataclasses
import functools

import jax
import jax.numpy as jnp
from jax.experimental import pallas as pl
from jax.experimental.pallas import tpu as pltpu
from jax.experimental.pallas import tpu_sc as plsc

N = 4096
D = 32
K = 16
KP1 = K + 1
NH = N // 2
BR = 512
CH = 512
GROUPS = CH // K


def _knn_body(emb_r_ref, embt_ref, knn_d_ref, knn_i_ref, *, row_base):
    emb_r = emb_r_ref[...]
    embt = embt_ref[...]
    sq_col = jnp.sum(emb_r * emb_r, axis=1, keepdims=True)
    sq_row = jnp.sum(embt * embt, axis=0, keepdims=True)
    dots = jax.lax.dot_general(
        emb_r, embt, (((1,), (0,)), ((), ())),
        preferred_element_type=jnp.float32)
    d2 = jnp.maximum(sq_col + sq_row - 2.0 * dots, 0.0)

    iota = jax.lax.broadcasted_iota(jnp.int32, (BR, N), 1)
    vmask = jnp.int32(-4096)
    maxi = jnp.int32(0x7FFFFFFF)
    row_id = (row_base + pl.program_id(0) * BR
              + jax.lax.broadcasted_iota(jnp.int32, (BR, N), 0))
    key = (jax.lax.bitcast_convert_type(d2, jnp.int32) & vmask) | iota
    key = jnp.where(iota == row_id, maxi, key)
    q = N // 4
    qa, qb = key[:, :q], key[:, q:2 * q]
    qc, qd = key[:, 2 * q:3 * q], key[:, 3 * q:]
    m1, x1 = jnp.minimum(qa, qb), jnp.maximum(qa, qb)
    m2, x2 = jnp.minimum(qc, qd), jnp.maximum(qc, qd)
    r4 = jnp.minimum(m1, m2)
    s4 = jnp.where(m1 < m2, jnp.minimum(m2, x1), jnp.minimum(m1, x2))
    e = N // 8
    ra, rb = r4[:, :e], r4[:, e:]
    sa, sb = s4[:, :e], s4[:, e:]
    r8 = jnp.minimum(ra, rb)
    s8 = jnp.where(ra < rb, jnp.minimum(rb, sa), jnp.minimum(ra, sb))
    e2 = N // 16
    ra2, rb2 = r8[:, :e2], r8[:, e2:]
    sa2, sb2 = s8[:, :e2], s8[:, e2:]
    r16 = jnp.minimum(ra2, rb2)
    s16 = jnp.where(ra2 < rb2, jnp.minimum(rb2, sa2), jnp.minimum(ra2, sb2))
    e3 = N // 32
    ra3, rb3 = r16[:, :e3], r16[:, e3:]
    sa3, sb3 = s16[:, :e3], s16[:, e3:]
    r = jnp.minimum(ra3, rb3)
    s = jnp.where(ra3 < rb3, jnp.minimum(rb3, sa3), jnp.minimum(ra3, sb3))
    for t in range(K):
        m = jnp.min(r, axis=1, keepdims=True)
        knn_d_ref[:, t:t + 1] = jax.lax.bitcast_convert_type(
            m & vmask, jnp.float32)
        knn_i_ref[:, t:t + 1] = m & jnp.int32(0xFFF)
        if t < K - 1:
            r = jnp.where(r == m, jnp.where(s == m, maxi, s), r)


def _knn_call(emb_half, embt, row_base):
    return pl.pallas_call(
        functools.partial(_knn_body, row_base=row_base),
        grid=(NH // BR,),
        in_specs=[
            pl.BlockSpec((BR, D), lambda i: (i, 0)),
            pl.BlockSpec((D, N), lambda i: (0, 0)),
        ],
        out_specs=[
            pl.BlockSpec((BR, K), lambda i: (i, 0)),
            pl.BlockSpec((BR, K), lambda i: (i, 0)),
        ],
        out_shape=[
            jax.ShapeDtypeStruct((NH, K), jnp.float32),
            jax.ShapeDtypeStruct((NH, K), jnp.int32),
        ],
    )(emb_half, embt)


_GATHER_W = 256
DPAD = 128


def _sc_gather(emb_pad, idx_flat):
    mesh = plsc.VectorSubcoreMesh(core_axis_name="core",
                                  subcore_axis_name="subcore")
    cp = pltpu.CompilerParams()
    if "needs_layout_passes" in pltpu.CompilerParams.__dataclass_fields__:
        cp = dataclasses.replace(cp, needs_layout_passes=False)

    @functools.partial(
        pl.kernel,
        out_type=jax.ShapeDtypeStruct((NH * K, DPAD), jnp.float32),
        mesh=mesh,
        compiler_params=cp,
    )
    def gather_kernel(x_hbm, i_hbm, o_hbm):
        def body(i_vmem, o_vmem):
            pltpu.sync_copy(x_hbm.at[i_vmem.at[0]], o_vmem)

        pltpu.emit_pipeline(
            body,
            grid=(NH * K // _GATHER_W,),
            in_specs=[pl.BlockSpec((1, _GATHER_W), index_map=lambda i: (0, i))],
            out_specs=[pl.BlockSpec((_GATHER_W, DPAD), index_map=lambda i: (i, 0))],
            core_axis_name=("core", "subcore"),
            dimension_semantics=(pltpu.PARALLEL,),
        )(i_hbm, o_hbm)

    return gather_kernel(emb_pad, idx_flat)


def _inter_body(g_ref, knn_d_ref, refc_ref, refd_ref,
                curv_ref, knnsum_ref, refdsum_ref):
    step = pl.program_id(0)

    @pl.when(step == 0)
    def _init():
        curv_ref[...] = jnp.zeros_like(curv_ref)
        knnsum_ref[...] = jnp.zeros_like(knnsum_ref)
        refdsum_ref[...] = jnp.zeros_like(refdsum_ref)

    knn_d = jnp.sqrt(knn_d_ref[...] + 1e-12)
    a = jnp.mean(knn_d, axis=1, keepdims=True)
    refc = refc_ref[...]

    ri = jax.lax.broadcasted_iota(jnp.int32, (CH, CH), 0)
    ci = jax.lax.broadcasted_iota(jnp.int32, (CH, CH), 1)
    mask = (ri // K == ci // K) & (ci > ri)
    gi = jax.lax.broadcasted_iota(jnp.int32, (GROUPS, CH), 0)
    gf = jax.lax.broadcasted_iota(jnp.int32, (GROUPS, CH), 1)
    pmat = (gf // K == gi).astype(jnp.float32)

    denom = K * (K - 1) / 2.0
    ones_row = jnp.ones((1, D), jnp.float32)
    partial = jnp.zeros((1, 1), jnp.float32)
    for j in range(BR * K // CH):
        sub = g_ref[j * CH:(j + 1) * CH, :D]
        sub2 = sub * sub
        sq_col = jnp.sum(sub2, axis=1, keepdims=True)
        sq_row = jax.lax.dot_general(
            ones_row, sub2, (((1,), (1,)), ((), ())),
            preferred_element_type=jnp.float32)
        dots = jax.lax.dot_general(
            sub, sub, (((1,), (1,)), ((), ())),
            preferred_element_type=jnp.float32)
        d2 = sq_col + sq_row - 2.0 * dots
        dist = jnp.sqrt(jnp.maximum(d2, 0.0) + 1e-12)
        msum = jnp.sum(jnp.where(mask, dist, 0.0), axis=1, keepdims=True)
        gs = jax.lax.dot_general(
            pmat, msum, (((1,), (0,)), ((), ())),
            preferred_element_type=jnp.float32)
        b = gs / denom
        a_j = a[j * GROUPS:(j + 1) * GROUPS, :]
        refc_j = refc[j * GROUPS:(j + 1) * GROUPS, :]
        diff = b / (a_j + 1e-8) - refc_j
        partial = partial + jnp.sum(diff * diff, keepdims=True)

    curv_ref[...] += partial
    knnsum_ref[...] += jnp.sum(knn_d, keepdims=True)
    refdsum_ref[...] += jnp.sum(refd_ref[...], keepdims=True)


def _inter_call(gathered, knn_d, refc, refd):
    return pl.pallas_call(
        _inter_body,
        grid=(NH // BR,),
        in_specs=[
            pl.BlockSpec((BR * K, DPAD), lambda i: (i, 0)),
            pl.BlockSpec((BR, K), lambda i: (i, 0)),
            pl.BlockSpec((BR, 1), lambda i: (i, 0)),
            pl.BlockSpec((BR, K), lambda i: (i, 0)),
        ],
        out_specs=[
            pl.BlockSpec((1, 1), lambda i: (0, 0)),
            pl.BlockSpec((1, 1), lambda i: (0, 0)),
            pl.BlockSpec((1, 1), lambda i: (0, 0)),
        ],
        out_shape=[
            jax.ShapeDtypeStruct((1, 1), jnp.float32),
            jax.ShapeDtypeStruct((1, 1), jnp.float32),
            jax.ShapeDtypeStruct((1, 1), jnp.float32),
        ],
    )(gathered, knn_d, refc, refd)


def kernel(embeddings, ref_curvature, ref_distances):
    emb = embeddings.astype(jnp.float32)
    embt = jnp.transpose(emb)
    emb_pad = jnp.pad(emb, ((0, 0), (0, DPAD - D)))
    refc = ref_curvature.reshape(N, 1)

    knn = [_knn_call(emb[h * NH:(h + 1) * NH], embt, h * NH) for h in range(2)]
    gathered = [_sc_gather(emb_pad, knn[h][1].reshape(1, NH * K))
                for h in range(2)]
    curv_sq = jnp.float32(0.0)
    knn_sum = jnp.float32(0.0)
    refd_sum = jnp.float32(0.0)
    for h in range(2):
        c, ks, rs = _inter_call(
            gathered[h], knn[h][0], refc[h * NH:(h + 1) * NH],
            ref_distances[h * NH:(h + 1) * NH])
        curv_sq = curv_sq + c[0, 0]
        knn_sum = knn_sum + ks[0, 0]
        refd_sum = refd_sum + rs[0, 0]

    curvature_loss = curv_sq / N
    scale_loss = (knn_sum / (N * K) - refd_sum / (N * K)) ** 2
    return curvature_loss + 0.1 * scale_loss
